# baseline (speedup 1.0000x reference)
import functools

import numpy as np
import jax
import jax.numpy as jnp
from jax import lax
from jax.experimental import pallas as pl
from jax.experimental.pallas import tpu as pltpu
from jax.experimental.pallas import tpu_sc as plsc

L = 16
NC, NS = 2, 16
NW = NC * NS


def _positional_encoding(seq_len, d_model):
    pos = np.arange(seq_len)[:, np.newaxis]
    dim = np.arange(d_model)[np.newaxis, :]
    angles = pos / np.power(10000, 2 * (dim // 2) / d_model)
    pe = np.zeros(angles.shape)
    pe[:, 0::2] = np.sin(angles[:, 0::2])
    pe[:, 1::2] = np.cos(angles[:, 1::2])
    return pe.astype(np.float32)


@functools.partial(jax.jit, static_argnums=(3, 4))
def _embed(idx_flat, table, pe, seq, d):
    ntot = idx_flat.shape[0]
    bb = (ntot // seq) // NW
    dt_n = d // 8
    bt_n = NW
    jg_n = bb // L

    mesh = plsc.VectorSubcoreMesh(core_axis_name="c", subcore_axis_name="s")

    @functools.partial(
        pl.kernel,
        mesh=mesh,
        compiler_params=pltpu.CompilerParams(
            needs_layout_passes=False, use_tc_tiling_on_sc=False
        ),
        out_type=jax.ShapeDtypeStruct((seq, dt_n, bt_n, 8, bb), jnp.float32),
        scratch_types=[
            pltpu.VMEM((bb * seq,), jnp.int32),
            pltpu.VMEM((seq * bb,), jnp.int32),
            pltpu.VMEM((seq, d), jnp.float32),
            pltpu.VMEM((2, bb, d), jnp.float32),
            pltpu.VMEM((2, d, bb), jnp.float32),
            pltpu.SemaphoreType.DMA,
            pltpu.SemaphoreType.DMA,
            pltpu.SemaphoreType.DMA,
            pltpu.SemaphoreType.DMA,
        ],
    )
    def body(idx_hbm, table_hbm, pe_hbm, out_hbm,
             idxraw, idxt, pe_v, gbuf, tbuf, gsemA, gsemB, osemA, osemB):
        w = lax.axis_index("s") * NC + lax.axis_index("c")
        b0 = w * bb

        pltpu.sync_copy(idx_hbm.at[pl.ds(b0 * seq, bb * seq)], idxraw)
        pltpu.sync_copy(pe_hbm, pe_v)

        lanes = lax.broadcasted_iota(jnp.int32, (L,), 0)
        row16 = [jg * L + lanes for jg in range(jg_n)]

        def tr_idx(s, _):
            for jg in range(jg_n):
                src = (jg * L + lanes) * seq + s
                v = plsc.load_gather(idxraw, [src])
                idxt[pl.ds(s * bb + jg * L, L)] = v
            return 0

        lax.fori_loop(0, seq, tr_idx, 0)

        def gather_desc(s, buf, sem):
            return pltpu.make_async_copy(
                table_hbm.at[idxt.at[pl.ds(s * bb, bb)]], gbuf.at[buf], sem
            )

        def out_descs(s, buf, sem):
            return [
                pltpu.make_async_copy(
                    tbuf.at[buf, pl.ds(dt * 8, 8)], out_hbm.at[s, dt, w], sem
                )
                for dt in range(dt_n)
            ]

        def compute(s, buf):
            def pscan(jg, acc):
                return acc | (idxt[pl.ds(s * bb + jg * L, L)] == 0)

            m_any = lax.fori_loop(0, jg_n, pscan, jnp.zeros((L,), jnp.bool_))
            npad = plsc.all_reduce_population_count(m_any)[0]

            @pl.when(npad > 0)
            def _():
                zeros = jnp.zeros((L,), jnp.float32)
                for jg in range(jg_n):
                    m = idxt[pl.ds(s * bb + jg * L, L)] == 0
                    for j in range(d):
                        colj = jnp.full((L,), j, jnp.int32)
                        plsc.store_scatter(
                            gbuf.at[buf], [row16[jg], colj], zeros, mask=m
                        )

            s16 = jnp.full((L,), s, jnp.int32)

            def col(dd, _):
                d16 = jnp.full((L,), dd, jnp.int32)
                pe16 = plsc.load_gather(pe_v, [s16, d16])
                for jg in range(jg_n):
                    g16 = plsc.load_gather(gbuf.at[buf], [row16[jg], d16])
                    tbuf[buf, dd, pl.ds(jg * L, L)] = g16 + pe16
                return 0

            lax.fori_loop(0, d, col, 0)

        gather_desc(0, 0, gsemA).start()

        def step(k, _):
            s0 = 2 * k
            s1 = 2 * k + 1
            gather_desc(s1, 1, gsemB).start()
            gather_desc(s0, 0, gsemA).wait()

            @pl.when(k > 0)
            def _():
                for dsc in out_descs(s0, 0, osemA):
                    dsc.wait()

            compute(s0, 0)
            for dsc in out_descs(s0, 0, osemA):
                dsc.start()

            @pl.when(k + 1 < seq // 2)
            def _():
                gather_desc(s0 + 2, 0, gsemA).start()

            gather_desc(s1, 1, gsemB).wait()

            @pl.when(k > 0)
            def _():
                for dsc in out_descs(s1, 1, osemB):
                    dsc.wait()

            compute(s1, 1)
            for dsc in out_descs(s1, 1, osemB):
                dsc.start()

            return 0

        lax.fori_loop(0, seq // 2, step, 0)

        for dsc in out_descs(seq - 2, 0, osemA):
            dsc.wait()
        for dsc in out_descs(seq - 1, 1, osemB):
            dsc.wait()

    return body(idx_flat, table, pe)


def kernel(input, table):
    b, s = input.shape
    v, d = table.shape
    idx_flat = input.reshape(-1).astype(jnp.int32)
    pe = jnp.asarray(_positional_encoding(s, d))
    out5 = _embed(idx_flat, table, pe, s, d)
    return out5.transpose(2, 4, 0, 1, 3).reshape(b, s, d)


# speedup vs baseline: 1.3088x; 1.3088x over previous
import functools

import numpy as np
import jax
import jax.numpy as jnp
from jax import lax
from jax.experimental import pallas as pl
from jax.experimental.pallas import tpu as pltpu
from jax.experimental.pallas import tpu_sc as plsc

L = 16
NC, NS = 2, 16
NW = NC * NS


def _positional_encoding(seq_len, d_model):
    pos = np.arange(seq_len)[:, np.newaxis]
    dim = np.arange(d_model)[np.newaxis, :]
    angles = pos / np.power(10000, 2 * (dim // 2) / d_model)
    pe = np.zeros(angles.shape)
    pe[:, 0::2] = np.sin(angles[:, 0::2])
    pe[:, 1::2] = np.cos(angles[:, 1::2])
    return pe.astype(np.float32)


@functools.partial(jax.jit, static_argnums=(3, 4))
def _embed(idx_flat, table, pe, seq, d):
    ntot = idx_flat.shape[0]
    bb = (ntot // seq) // NW
    dt_n = d // 8
    bt_n = NW
    jg_n = bb // L

    mesh = plsc.VectorSubcoreMesh(core_axis_name="c", subcore_axis_name="s")

    @functools.partial(
        pl.kernel,
        mesh=mesh,
        compiler_params=pltpu.CompilerParams(
            needs_layout_passes=False, use_tc_tiling_on_sc=False
        ),
        out_type=jax.ShapeDtypeStruct((seq, dt_n, bt_n, 8, bb), jnp.float32),
        scratch_types=[
            pltpu.VMEM((bb * seq,), jnp.int32),
            pltpu.VMEM((seq * bb,), jnp.int32),
            pltpu.VMEM((seq, d), jnp.float32),
            pltpu.VMEM((2, bb, d), jnp.float32),
            pltpu.VMEM((2, d, bb), jnp.float32),
            pltpu.SemaphoreType.DMA,
            pltpu.SemaphoreType.DMA,
            pltpu.SemaphoreType.DMA,
            pltpu.SemaphoreType.DMA,
        ],
    )
    def body(idx_hbm, table_hbm, pe_hbm, out_hbm,
             idxraw, idxt, pe_v, gbuf, tbuf, gsemA, gsemB, osemA, osemB):
        w = lax.axis_index("s") * NC + lax.axis_index("c")
        b0 = w * bb

        pltpu.sync_copy(idx_hbm.at[pl.ds(b0 * seq, bb * seq)], idxraw)
        pltpu.sync_copy(pe_hbm, pe_v)

        lanes = lax.broadcasted_iota(jnp.int32, (L,), 0)
        row16 = [jg * L + lanes for jg in range(jg_n)]

        def tr_idx(s, _):
            for jg in range(jg_n):
                src = (jg * L + lanes) * seq + s
                v = plsc.load_gather(idxraw, [src])
                idxt[pl.ds(s * bb + jg * L, L)] = v
            return 0

        lax.fori_loop(0, seq, tr_idx, 0)

        def gather_desc(s, buf, sem):
            return pltpu.make_async_copy(
                table_hbm.at[idxt.at[pl.ds(s * bb, bb)]], gbuf.at[buf], sem
            )

        def out_descs(s, buf, sem):
            return [
                pltpu.make_async_copy(
                    tbuf.at[buf, pl.ds(dt * 8, 8)], out_hbm.at[s, dt, w], sem
                )
                for dt in range(dt_n)
            ]

        def compute(s, buf):
            def pscan(jg, acc):
                return acc | (idxt[pl.ds(s * bb + jg * L, L)] == 0)

            m_any = lax.fori_loop(0, jg_n, pscan, jnp.zeros((L,), jnp.bool_))
            npad = plsc.all_reduce_population_count(m_any)[0]

            @pl.when(npad > 0)
            def _():
                zeros = jnp.zeros((L,), jnp.float32)
                for jg in range(jg_n):
                    m = idxt[pl.ds(s * bb + jg * L, L)] == 0
                    for j in range(d):
                        colj = jnp.full((L,), j, jnp.int32)
                        plsc.store_scatter(
                            gbuf.at[buf], [row16[jg], colj], zeros, mask=m
                        )

            s16 = jnp.full((L,), s, jnp.int32)

            def col(dd, _):
                d16 = jnp.full((L,), dd, jnp.int32)
                pe16 = plsc.load_gather(pe_v, [s16, d16])
                g = [
                    plsc.load_gather(gbuf.at[buf], [row16[jg], d16])
                    for jg in range(jg_n)
                ]
                acc = [g16 + pe16 for g16 in g]
                for jg in range(jg_n):
                    tbuf[buf, dd, pl.ds(jg * L, L)] = acc[jg]
                return 0

            lax.fori_loop(0, d, col, 0)

        gather_desc(0, 0, gsemA).start()

        def step(k, _):
            s0 = 2 * k
            s1 = 2 * k + 1
            gather_desc(s1, 1, gsemB).start()
            gather_desc(s0, 0, gsemA).wait()

            @pl.when(k > 0)
            def _():
                for dsc in out_descs(s0, 0, osemA):
                    dsc.wait()

            compute(s0, 0)
            for dsc in out_descs(s0, 0, osemA):
                dsc.start()

            @pl.when(k + 1 < seq // 2)
            def _():
                gather_desc(s0 + 2, 0, gsemA).start()

            gather_desc(s1, 1, gsemB).wait()

            @pl.when(k > 0)
            def _():
                for dsc in out_descs(s1, 1, osemB):
                    dsc.wait()

            compute(s1, 1)
            for dsc in out_descs(s1, 1, osemB):
                dsc.start()

            return 0

        lax.fori_loop(0, seq // 2, step, 0)

        for dsc in out_descs(seq - 2, 0, osemA):
            dsc.wait()
        for dsc in out_descs(seq - 1, 1, osemB):
            dsc.wait()

    return body(idx_flat, table, pe)


def kernel(input, table):
    b, s = input.shape
    v, d = table.shape
    idx_flat = input.reshape(-1).astype(jnp.int32)
    pe = jnp.asarray(_positional_encoding(s, d))
    out5 = _embed(idx_flat, table, pe, s, d)
    return out5.transpose(2, 4, 0, 1, 3).reshape(b, s, d)
